# baseline (speedup 1.0000x reference)
import functools

import jax
import jax.numpy as jnp
from jax import lax
from jax.experimental import pallas as pl
from jax.experimental.pallas import tpu as pltpu
from jax.experimental.pallas import tpu_sc as plsc

TOPK = 32
BLK = 2048
CHUNK = 128
SUB = 32
QG = 8

_NEG_INF = float("-inf")
_BIG_I32 = 2**31 - 1


def _sims_chunkmax_body(q_ref, k_ref, sims_ref, cmax_ref, *, n_mem):
    j = pl.program_id(0)
    s = lax.dot_general(
        q_ref[...], k_ref[...],
        (((1,), (1,)), ((), ())),
        preferred_element_type=jnp.float32,
    )
    n_q = s.shape[0]
    s3 = s.reshape(n_q // QG, QG, BLK)
    for c in range(BLK // CHUNK):
        sims_ref[:, c, :, :] = lax.slice(
            s3, (0, 0, c * CHUNK), (n_q // QG, QG, (c + 1) * CHUNK))
    col = lax.broadcasted_iota(jnp.int32, (n_q, BLK), 1) + j * BLK
    sm = jnp.where(col < n_mem, s, _NEG_INF)
    parts = []
    for c in range(BLK // CHUNK):
        piece = lax.slice(sm, (0, c * CHUNK), (n_q, (c + 1) * CHUNK))
        parts.append(jnp.max(piece, axis=1, keepdims=True))
    cmax_ref[0, :, :] = jnp.concatenate(parts, axis=1)


def _chunk_select_body(cmaxt_ref, rows_ref, sel_ref, *, n_chunks):
    run = cmaxt_ref[...]
    n_q = run.shape[1]
    cid = lax.broadcasted_iota(jnp.int32, run.shape, 0)
    outs = []
    for _ in range(TOPK):
        m = jnp.max(run, axis=0, keepdims=True)
        elig = run == m
        gi = jnp.min(jnp.where(elig, cid, _BIG_I32), axis=0, keepdims=True)
        outs.append(gi)
        run = jnp.where(cid == gi, _NEG_INF, run)
    sel = jnp.concatenate(outs, axis=0)
    qcol = lax.broadcasted_iota(jnp.int32, (TOPK, n_q), 1)
    rows_ref[...] = ((qcol // QG) * (n_chunks * QG) + sel * QG
                     + (qcol % QG))
    sel_ref[...] = sel


def _submax_body(cand_ref, sel_ref, smax_ref, *, n_mem):
    nr, chunk_w = cand_ref.shape
    selc = sel_ref[...]
    off = lax.broadcasted_iota(jnp.int32, (nr, chunk_w), 1)
    gidx = selc * chunk_w + off
    run = jnp.where(gidx < n_mem, cand_ref[...], _NEG_INF)
    parts = []
    for t in range(chunk_w // SUB):
        piece = lax.slice(run, (0, t * SUB), (nr, (t + 1) * SUB))
        parts.append(jnp.max(piece, axis=1, keepdims=True))
    smax_ref[...] = jnp.concatenate(parts, axis=1)


def _sub_select_body(smaxt_ref, selt_ref, rows2_ref, sub_ref):
    n_slots, n_q = smaxt_ref.shape
    per = CHUNK // SUB
    run = smaxt_ref[...]
    selt = selt_ref[...]
    sub_id = (selt.reshape(TOPK, 1, n_q) * per
              + lax.broadcasted_iota(jnp.int32, (TOPK, per, n_q), 1)
              ).reshape(n_slots, n_q)
    slot = lax.broadcasted_iota(jnp.int32, (n_slots, n_q), 0)
    qcol = lax.broadcasted_iota(jnp.int32, (TOPK, n_q), 1)
    rows_out, sub_out = [], []
    for _ in range(TOPK):
        m = jnp.max(run, axis=0, keepdims=True)
        elig = run == m
        gi = jnp.min(jnp.where(elig, sub_id, _BIG_I32), axis=0,
                     keepdims=True)
        win = elig & (sub_id == gi)
        sl = jnp.min(jnp.where(win, slot, _BIG_I32), axis=0, keepdims=True)
        rows_out.append(sl)
        sub_out.append(gi)
        run = jnp.where(sub_id == gi, _NEG_INF, run)
    slots = jnp.concatenate(rows_out, axis=0)
    rows2_ref[...] = qcol * n_slots + slots
    sub_ref[...] = jnp.concatenate(sub_out, axis=0)


def _final_select_body(candt_ref, subt_ref, o_ref, *, n_mem):
    n_cand, n_q = candt_ref.shape
    subt = subt_ref[...]
    gidx = (subt.reshape(TOPK, 1, n_q) * SUB
            + lax.broadcasted_iota(jnp.int32, (TOPK, SUB, n_q), 1)
            ).reshape(n_cand, n_q)
    run = jnp.where(gidx < n_mem, candt_ref[...], _NEG_INF)
    outs = []
    for _ in range(TOPK):
        m = jnp.max(run, axis=0, keepdims=True)
        elig = run == m
        gi = jnp.min(jnp.where(elig, gidx, _BIG_I32), axis=0, keepdims=True)
        outs.append(gi)
        run = jnp.where(gidx == gi, _NEG_INF, run)
    o_ref[...] = jnp.concatenate(outs, axis=0)


def _sc_gather(tables, flat_idx, window=128, tc_tiling=True):
    n_idx = flat_idx.shape[0]
    idx2 = flat_idx.reshape(1, n_idx)
    mesh = plsc.VectorSubcoreMesh(
        core_axis_name="core", subcore_axis_name="subcore"
    )
    out_type = tuple(
        jax.ShapeDtypeStruct((n_idx, t.shape[1]), t.dtype) for t in tables
    )
    cp = pltpu.CompilerParams(use_tc_tiling_on_sc=tc_tiling)

    @functools.partial(pl.kernel, out_type=out_type, mesh=mesh,
                       compiler_params=cp)
    def gather_kernel(*refs):
        t_hbm = refs[:len(tables)]
        i_hbm = refs[len(tables)]
        o_hbm = refs[len(tables) + 1:]

        def body(i_vmem, *o_vmem):
            for t, o in zip(t_hbm, o_vmem):
                pltpu.sync_copy(t.at[i_vmem.at[0]], o)

        pltpu.emit_pipeline(
            body,
            grid=(n_idx // window,),
            in_specs=[pl.BlockSpec((1, window), lambda i: (0, i))],
            out_specs=[
                pl.BlockSpec((window, t.shape[1]), lambda i: (i, 0))
                for t in tables
            ],
            core_axis_name=("core", "subcore"),
            dimension_semantics=(pltpu.PARALLEL,),
        )(i_hbm, *o_hbm)

    outs = gather_kernel(*tables, idx2)
    return outs if isinstance(outs, (tuple, list)) else (outs,)


def kernel(q, k_memory, v_memory):
    n_q, d = q.shape
    n_mem = k_memory.shape[0]
    n_pad = (-n_mem) % BLK
    m_pad = n_mem + n_pad
    n_blocks = m_pad // BLK
    n_chunks = m_pad // CHUNK
    per = CHUNK // SUB
    k_pad = jnp.pad(k_memory, ((0, n_pad), (0, 0)))

    sims, cmax = pl.pallas_call(
        functools.partial(_sims_chunkmax_body, n_mem=n_mem),
        grid=(n_blocks,),
        in_specs=[
            pl.BlockSpec((n_q, d), lambda j: (0, 0)),
            pl.BlockSpec((BLK, d), lambda j: (j, 0)),
        ],
        out_specs=[
            pl.BlockSpec((n_q // QG, BLK // CHUNK, QG, CHUNK),
                         lambda j: (0, j, 0, 0)),
            pl.BlockSpec((1, n_q, BLK // CHUNK), lambda j: (j, 0, 0)),
        ],
        out_shape=[
            jax.ShapeDtypeStruct((n_q // QG, n_chunks, QG, CHUNK),
                                 jnp.float32),
            jax.ShapeDtypeStruct((n_blocks, n_q, BLK // CHUNK), jnp.float32),
        ],
    )(q, k_pad)
    cmaxt = cmax.transpose(0, 2, 1).reshape(n_chunks, n_q)

    chunk_rows, selt = pl.pallas_call(
        functools.partial(_chunk_select_body, n_chunks=n_chunks),
        out_shape=[
            jax.ShapeDtypeStruct((TOPK, n_q), jnp.int32),
            jax.ShapeDtypeStruct((TOPK, n_q), jnp.int32),
        ],
    )(cmaxt)

    sims_chunks = sims.reshape(n_q * n_chunks, CHUNK)
    (cand,) = _sc_gather((sims_chunks,), chunk_rows.T.reshape(-1))

    n_rows = n_q * TOPK
    d1_steps = 8
    (smax,) = pl.pallas_call(
        functools.partial(_submax_body, n_mem=n_mem),
        grid=(d1_steps,),
        in_specs=[
            pl.BlockSpec((n_rows // d1_steps, CHUNK), lambda i: (i, 0)),
            pl.BlockSpec((n_rows // d1_steps, 1), lambda i: (i, 0)),
        ],
        out_specs=[pl.BlockSpec((n_rows // d1_steps, per),
                                lambda i: (i, 0))],
        out_shape=[jax.ShapeDtypeStruct((n_rows, per), jnp.float32)],
    )(cand, selt.T.reshape(n_rows, 1))

    smaxt = smax.reshape(n_q, TOPK * per).T
    rows2, subt = pl.pallas_call(
        _sub_select_body,
        out_shape=[
            jax.ShapeDtypeStruct((TOPK, n_q), jnp.int32),
            jax.ShapeDtypeStruct((TOPK, n_q), jnp.int32),
        ],
    )(smaxt, selt)

    cand_subs = cand.reshape(n_rows * per, SUB)
    (cand2,) = _sc_gather((cand_subs,), rows2.T.reshape(-1),
                          tc_tiling=False)

    candt2 = cand2.reshape(n_q, TOPK, SUB).transpose(1, 2, 0).reshape(
        TOPK * SUB, n_q)
    (idxt,) = pl.pallas_call(
        functools.partial(_final_select_body, n_mem=n_mem),
        out_shape=[jax.ShapeDtypeStruct((TOPK, n_q), jnp.int32)],
    )(candt2, subt)

    flat_idx = idxt.T.reshape(-1)
    k_rows, v_rows = _sc_gather((k_memory, v_memory), flat_idx)
    return (k_rows, v_rows)
